# baseline (speedup 1.0000x reference)
import functools

import jax
import jax.numpy as jnp
from jax import lax
from jax.experimental import pallas as pl
from jax.experimental.pallas import tpu as pltpu
from jax.experimental.pallas import tpu_sc as plsc

_ROWS = 128
_COLS = 32768
_L = 16
_NW = 32
_RPW = _ROWS // _NW
_CH = 8192


def _sc_body(x_hbm, o_hbm, buf, sem):
    wid = lax.axis_index("s") * 2 + lax.axis_index("c")
    r0 = wid * _RPW
    last = jnp.full((_L,), _L - 1, jnp.int32)

    def do_chunk(c, carries):
        col = c * _CH
        pltpu.async_copy(
            x_hbm.at[pl.ds(r0, _RPW), pl.ds(col, _CH)], buf, sem
        ).wait()

        def vreg_step(i, cs):
            out = []
            for r in range(_RPW):
                v = buf[r, pl.ds(i * _L, _L)]
                y = plsc.cumsum(v) + cs[r]
                buf[r, pl.ds(i * _L, _L)] = y
                out.append(jnp.take_along_axis(y, last, axis=0))
            return tuple(out)

        carries = lax.fori_loop(0, _CH // _L, vreg_step, carries, unroll=4)
        pltpu.async_copy(
            buf, o_hbm.at[pl.ds(r0, _RPW), pl.ds(col, _CH)], sem
        ).wait()
        return carries

    zero = jnp.zeros((_L,), jnp.float32)
    lax.fori_loop(0, _COLS // _CH, do_chunk, (zero,) * _RPW)


def kernel(x):
    mesh = plsc.VectorSubcoreMesh(core_axis_name="c", subcore_axis_name="s")
    f = pl.kernel(
        _sc_body,
        out_type=jax.ShapeDtypeStruct((_ROWS, _COLS), jnp.float32),
        mesh=mesh,
        scratch_types=[
            pltpu.VMEM((_RPW, _CH), jnp.float32),
            pltpu.SemaphoreType.DMA,
        ],
        compiler_params=pltpu.CompilerParams(needs_layout_passes=False),
    )
    return f(x)


# speedup vs baseline: 2.3057x; 2.3057x over previous
import jax
import jax.numpy as jnp
from jax import lax
from jax.experimental import pallas as pl
from jax.experimental.pallas import tpu as pltpu
from jax.experimental.pallas import tpu_sc as plsc

_ROWS = 128
_COLS = 32768
_TC_ROWS = 96
_SC_ROWS = _ROWS - _TC_ROWS
_BLOCK = 4096
_CHUNK = 128
_NCHUNK = _BLOCK // _CHUNK
_L = 16
_NW = 32


def _cumsum_block(x_ref, o_ref, carry_ref):
    @pl.when(pl.program_id(0) == 0)
    def _init():
        carry_ref[...] = jnp.zeros_like(carry_ref)

    row = jax.lax.broadcasted_iota(jnp.int32, (_CHUNK, _CHUNK), 0)
    col = jax.lax.broadcasted_iota(jnp.int32, (_CHUNK, _CHUNK), 1)
    tri = (row <= col).astype(jnp.bfloat16)

    xb = x_ref[...]
    hi_b = xb.astype(jnp.bfloat16)
    lo_b = (xb - hi_b.astype(jnp.float32)).astype(jnp.bfloat16)

    def mm(a, b):
        return jax.lax.dot_general(
            a, b, (((1,), (0,)), ((), ())),
            preferred_element_type=jnp.float32,
        )

    cs = []
    for j in range(_NCHUNK):
        sl = slice(j * _CHUNK, (j + 1) * _CHUNK)
        cs.append(mm(hi_b[:, sl], tri) + mm(lo_b[:, sl], tri))

    carry = carry_ref[:, 0:1]
    offs = [carry]
    tot = [c[:, _CHUNK - 1:_CHUNK] for c in cs]
    pre = [None] * _NCHUNK
    for j in range(_NCHUNK):
        pre[j] = tot[j] if j == 0 else pre[j - 1] + tot[j]
    for j in range(1, _NCHUNK):
        offs.append(carry + pre[j - 1])

    for j in range(_NCHUNK):
        o_ref[:, j * _CHUNK:(j + 1) * _CHUNK] = cs[j] + offs[j]
    carry_ref[:, 0:1] = carry + pre[_NCHUNK - 1]


def _tc_part(x):
    return pl.pallas_call(
        _cumsum_block,
        grid=(_COLS // _BLOCK,),
        in_specs=[pl.BlockSpec((_TC_ROWS, _BLOCK), lambda i: (0, i))],
        out_specs=pl.BlockSpec((_TC_ROWS, _BLOCK), lambda i: (0, i)),
        out_shape=jax.ShapeDtypeStruct((_TC_ROWS, _COLS), x.dtype),
        scratch_shapes=[pltpu.VMEM((_TC_ROWS, 128), jnp.float32)],
    )(x)


def _sc_body(x_hbm, o_hbm, buf, sem):
    wid = lax.axis_index("s") * 2 + lax.axis_index("c")
    row = _TC_ROWS + wid
    pltpu.async_copy(x_hbm.at[row], buf, sem).wait()

    def vreg_step(i, carry):
        v = buf[pl.ds(i * _L, _L)]
        s = plsc.cumsum(v)
        buf[pl.ds(i * _L, _L)] = s + carry
        return carry + jnp.sum(v)

    lax.fori_loop(0, _COLS // _L, vreg_step, jnp.float32(0.0), unroll=8)
    pltpu.async_copy(buf, o_hbm.at[wid], sem).wait()


def _sc_part(x):
    mesh = plsc.VectorSubcoreMesh(core_axis_name="c", subcore_axis_name="s")
    f = pl.kernel(
        _sc_body,
        out_type=jax.ShapeDtypeStruct((_SC_ROWS, _COLS), jnp.float32),
        mesh=mesh,
        scratch_types=[
            pltpu.VMEM((_COLS,), jnp.float32),
            pltpu.SemaphoreType.DMA,
        ],
        compiler_params=pltpu.CompilerParams(needs_layout_passes=False),
    )
    return f(x)


def kernel(x):
    return jnp.concatenate([_tc_part(x), _sc_part(x)], axis=0)


# speedup vs baseline: 6.3530x; 2.7553x over previous
import jax
import jax.numpy as jnp
from jax.experimental import pallas as pl
from jax.experimental.pallas import tpu as pltpu

_ROWS = 128
_BLOCK = 8192
_CHUNK = 128
_NCHUNK = _BLOCK // _CHUNK


def _cumsum_block(x_ref, o_ref, carry_ref):
    @pl.when(pl.program_id(0) == 0)
    def _init():
        carry_ref[...] = jnp.zeros_like(carry_ref)

    row = jax.lax.broadcasted_iota(jnp.int32, (_CHUNK, _CHUNK), 0)
    col = jax.lax.broadcasted_iota(jnp.int32, (_CHUNK, _CHUNK), 1)
    tri = (row <= col).astype(jnp.bfloat16)

    xb = x_ref[...]
    hi_b = xb.astype(jnp.bfloat16)
    lo_b = (xb - hi_b.astype(jnp.float32)).astype(jnp.bfloat16)

    def mm(a, b):
        return jax.lax.dot_general(
            a, b, (((1,), (0,)), ((), ())),
            preferred_element_type=jnp.float32,
        )

    cs = []
    for j in range(_NCHUNK):
        sl = slice(j * _CHUNK, (j + 1) * _CHUNK)
        cs.append(mm(hi_b[:, sl], tri) + mm(lo_b[:, sl], tri))

    carry = carry_ref[:, 0:1]
    offs = [carry]
    tot = [c[:, _CHUNK - 1:_CHUNK] for c in cs]
    pre = [None] * _NCHUNK
    for j in range(_NCHUNK):
        pre[j] = tot[j] if j == 0 else pre[j - 1] + tot[j]
    for j in range(1, _NCHUNK):
        offs.append(carry + pre[j - 1])

    for j in range(_NCHUNK):
        o_ref[:, j * _CHUNK:(j + 1) * _CHUNK] = cs[j] + offs[j]
    carry_ref[:, 0:1] = carry + pre[_NCHUNK - 1]


def kernel(x):
    rows, cols = x.shape
    grid = cols // _BLOCK
    return pl.pallas_call(
        _cumsum_block,
        grid=(grid,),
        in_specs=[pl.BlockSpec((rows, _BLOCK), lambda i: (0, i))],
        out_specs=pl.BlockSpec((rows, _BLOCK), lambda i: (0, i)),
        out_shape=jax.ShapeDtypeStruct((rows, cols), x.dtype),
        scratch_shapes=[pltpu.VMEM((rows, 128), jnp.float32)],
    )(x)
